# baseline (speedup 1.0000x reference)
import functools

import jax
import jax.numpy as jnp
from jax import lax
from jax.experimental import pallas as pl
from jax.experimental.pallas import tpu as pltpu
from jax.experimental.pallas import tpu_sc as plsc

NB_TOKENS = 1000000
DIM = 64
ROWS, COLS = 16384, 26
B = ROWS * COLS
NC, NS = 2, 16
NW = NC * NS
BPW = B // NW
CHUNK = 128
NCHUNK = BPW // CHUNK

_mesh = plsc.VectorSubcoreMesh(core_axis_name="c", subcore_axis_name="s")


@functools.partial(
    pl.kernel,
    mesh=_mesh,
    out_type=jax.ShapeDtypeStruct((B, DIM), jnp.float32),
    scratch_types=[
        pltpu.VMEM((NCHUNK, CHUNK), jnp.int32),
        pltpu.VMEM((CHUNK, DIM), jnp.float32),
        pltpu.SemaphoreType.DMA,
    ],
    compiler_params=pltpu.CompilerParams(use_tc_tiling_on_sc=False),
)
def _emb_lookup(idx_hbm, table_hbm, out_hbm, idx_v, rows_v, gsem):
    wid = lax.axis_index("s") * NC + lax.axis_index("c")
    pltpu.sync_copy(idx_hbm.at[wid], idx_v)
    base = wid * BPW

    def chunk_body(j, carry):
        pltpu.async_copy(table_hbm.at[idx_v.at[j]], rows_v, gsem).wait()
        pltpu.sync_copy(rows_v, out_hbm.at[pl.ds(base + j * CHUNK, CHUNK)])
        return carry

    lax.fori_loop(0, NCHUNK, chunk_body, 0)


def kernel(x, weight):
    idx = x.reshape(NW, NCHUNK, CHUNK).astype(jnp.int32)
    out = _emb_lookup(idx, weight)
    return out.reshape(ROWS, COLS, DIM)


# speedup vs baseline: 1.0775x; 1.0775x over previous
import functools

import jax
import jax.numpy as jnp
from jax import lax
from jax.experimental import pallas as pl
from jax.experimental.pallas import tpu as pltpu
from jax.experimental.pallas import tpu_sc as plsc

NB_TOKENS = 1000000
DIM = 64
ROWS, COLS = 16384, 26
B = ROWS * COLS
NC, NS = 2, 16
NW = NC * NS
BPW = B // NW
CHUNK = 128
NCHUNK = BPW // CHUNK
NBUF = 4
LAG = 2

_mesh = plsc.VectorSubcoreMesh(core_axis_name="c", subcore_axis_name="s")


@functools.partial(
    pl.kernel,
    mesh=_mesh,
    out_type=jax.ShapeDtypeStruct((B, DIM), jnp.float32),
    scratch_types=[
        pltpu.VMEM((NCHUNK, CHUNK), jnp.int32),
        pltpu.VMEM((NBUF, CHUNK, DIM), jnp.float32),
        pltpu.SemaphoreType.DMA,
        pltpu.SemaphoreType.DMA,
    ],
    compiler_params=pltpu.CompilerParams(use_tc_tiling_on_sc=False),
)
def _emb_lookup(idx_hbm, table_hbm, out_hbm, idx_v, rows_v, gsem, ssem):
    wid = lax.axis_index("s") * NC + lax.axis_index("c")
    pltpu.sync_copy(idx_hbm.at[wid], idx_v)
    base = wid * BPW

    def start_gather(j):
        pltpu.async_copy(table_hbm.at[idx_v.at[j]], rows_v.at[j % NBUF], gsem)

    def wait_gather(j):
        pltpu.make_async_copy(
            table_hbm.at[idx_v.at[j]], rows_v.at[j % NBUF], gsem).wait()

    def start_scatter(j):
        pltpu.async_copy(
            rows_v.at[j % NBUF], out_hbm.at[pl.ds(base + j * CHUNK, CHUNK)],
            ssem)

    def wait_scatter(j):
        pltpu.make_async_copy(
            rows_v.at[j % NBUF], out_hbm.at[pl.ds(base + j * CHUNK, CHUNK)],
            ssem).wait()

    for j in range(LAG):
        start_gather(j)
    for j in range(LAG):
        start_gather(j + LAG)
        wait_gather(j)
        start_scatter(j)

    def body(j, carry):
        wait_scatter(j - LAG)
        start_gather(j + LAG)
        wait_gather(j)
        start_scatter(j)
        return carry

    lax.fori_loop(LAG, NCHUNK - LAG, body, 0)

    for j in range(NCHUNK - LAG, NCHUNK):
        wait_scatter(j - LAG)
        wait_gather(j)
        start_scatter(j)
    for j in range(NCHUNK - LAG, NCHUNK):
        wait_scatter(j)


def kernel(x, weight):
    idx = x.reshape(NW, NCHUNK, CHUNK).astype(jnp.int32)
    out = _emb_lookup(idx, weight)
    return out.reshape(ROWS, COLS, DIM)


# speedup vs baseline: 1.1227x; 1.0419x over previous
import functools

import jax
import jax.numpy as jnp
from jax import lax
from jax.experimental import pallas as pl
from jax.experimental.pallas import tpu as pltpu
from jax.experimental.pallas import tpu_sc as plsc

NB_TOKENS = 1000000
DIM = 64
ROWS, COLS = 16384, 26
B = ROWS * COLS
NC, NS = 2, 16
NW = NC * NS
BPW = B // NW
CHUNK = 128
NCHUNK = BPW // CHUNK
NBUF = 4
LAG = 2

_mesh = plsc.VectorSubcoreMesh(core_axis_name="c", subcore_axis_name="s")


@functools.partial(
    pl.kernel,
    mesh=_mesh,
    out_type=jax.ShapeDtypeStruct((B, DIM), jnp.float32),
    scratch_types=[
        pltpu.VMEM((NCHUNK, CHUNK), jnp.int32),
        pltpu.VMEM((NBUF, CHUNK, DIM), jnp.float32),
        pltpu.SemaphoreType.DMA,
        pltpu.SemaphoreType.DMA,
    ],
    compiler_params=pltpu.CompilerParams(use_tc_tiling_on_sc=False),
)
def _emb_lookup(idx_hbm, table_hbm, out_hbm, idx_v, rows_v, gsem, ssem):
    wid = lax.axis_index("s") * NC + lax.axis_index("c")
    pltpu.sync_copy(idx_hbm.at[wid], idx_v)
    base = wid * BPW

    def start_gather(j):
        pltpu.async_copy(table_hbm.at[idx_v.at[j]], rows_v.at[j % NBUF], gsem)

    def wait_gather(j):
        pltpu.make_async_copy(
            table_hbm.at[idx_v.at[j]], rows_v.at[j % NBUF], gsem).wait()

    def start_scatter(j):
        pltpu.async_copy(
            rows_v.at[j % NBUF], out_hbm.at[pl.ds(base + j * CHUNK, CHUNK)],
            ssem)

    def wait_scatter(j):
        pltpu.make_async_copy(
            rows_v.at[j % NBUF], out_hbm.at[pl.ds(base + j * CHUNK, CHUNK)],
            ssem).wait()

    for j in range(LAG):
        start_gather(j)
    for j in range(LAG):
        start_gather(j + LAG)
        wait_gather(j)
        start_scatter(j)

    def body(j, carry):
        wait_scatter(j - LAG)
        start_gather(j + LAG)
        wait_gather(j)
        start_scatter(j)
        return carry

    lax.fori_loop(LAG, NCHUNK - LAG, body, 0)

    for j in range(NCHUNK - LAG, NCHUNK):
        wait_scatter(j - LAG)
        wait_gather(j)
        start_scatter(j)
    for j in range(NCHUNK - LAG, NCHUNK):
        wait_scatter(j)


def kernel(x, weight):
    idx = jnp.swapaxes(x, 0, 1).reshape(NW, NCHUNK, CHUNK).astype(jnp.int32)
    out = _emb_lookup(idx, weight)
    return out.reshape(COLS, ROWS, DIM).transpose(1, 0, 2)
